# baseline (speedup 1.0000x reference)
import functools

import jax
import jax.numpy as jnp
from jax import lax
from jax.experimental import pallas as pl
from jax.experimental.pallas import tpu as pltpu
from jax.experimental.pallas import tpu_sc as plsc

B = 16384
INPUT_DIM = 39
NUM_CAT = 26
VOCAB = 100000
EMB = 32
NUM_NUM = 13
HALF = NUM_CAT // 2

NC = 2
NS = 16
NW = NC * NS
SAMPLES_PER_TILE = B // NW
CHUNK = 128
NUM_CHUNKS = SAMPLES_PER_TILE // CHUNK
GROUPS = CHUNK // 16

CBLK = 4096
NBLK = (VOCAB + CBLK - 1) // CBLK


def _project_body(w_ref, t_ref, p_ref):
    p_ref[...] = jax.lax.dot_general(
        w_ref[...],
        t_ref[...],
        dimension_numbers=(((1,), (0,)), ((), ())),
        precision=jax.lax.Precision.DEFAULT,
        preferred_element_type=jnp.float32,
    )


def _gather_body(first, xt_hbm, p_hbm, pin_hbm, wb_hbm, out_hbm,
                 xv, idxv, gv, wv, outv, sem):
    wid = lax.axis_index("s") * NC + lax.axis_index("c")
    tile_base = pl.multiple_of(wid * SAMPLES_PER_TILE, SAMPLES_PER_TILE)
    col0 = NUM_NUM if first else NUM_NUM + HALF

    pltpu.sync_copy(wb_hbm, wv)

    def chunk_body(c, carry):
        base = pl.multiple_of(tile_base + c * CHUNK, CHUNK)

        pltpu.sync_copy(xt_hbm.at[:, pl.ds(base, CHUNK)], xv)
        if not first:
            pltpu.sync_copy(pin_hbm.at[pl.ds(base, CHUNK)], outv)

        def idx_body(f, carry):
            for g in range(GROUPS):
                v = xv[col0 + f, pl.ds(g * 16, 16)].astype(jnp.int32)
                idxv[f, pl.ds(g * 16, 16)] = v + f * VOCAB
            return carry

        lax.fori_loop(0, HALF, idx_body, 0)

        copies = [
            pltpu.async_copy(p_hbm.at[idxv.at[f]], gv.at[f], sem)
            for f in range(HALF)
        ]
        for cp in copies:
            cp.wait()

        wtail = wv[pl.ds(0, 16)]
        for g in range(GROUPS):
            acc = jnp.zeros((16,), jnp.float32)
            for f in range(HALF):
                acc = acc + gv[f, pl.ds(g * 16, 16)]
            if first:
                for i in range(NUM_NUM):
                    acc = acc + xv[i, pl.ds(g * 16, 16)] * wtail[i]
                outv[pl.ds(g * 16, 16)] = acc + wtail[NUM_NUM]
            else:
                outv[pl.ds(g * 16, 16)] = outv[pl.ds(g * 16, 16)] + acc

        pltpu.sync_copy(outv, out_hbm.at[pl.ds(base, CHUNK)])
        return carry

    lax.fori_loop(0, NUM_CHUNKS, chunk_body, 0)


def _make_gather(first):
    mesh = plsc.VectorSubcoreMesh(
        core_axis_name="c", subcore_axis_name="s", num_cores=NC, num_subcores=NS
    )
    return pl.kernel(
        functools.partial(_gather_body, first),
        out_type=jax.ShapeDtypeStruct((B,), jnp.float32),
        mesh=mesh,
        compiler_params=pltpu.CompilerParams(
            needs_layout_passes=False, use_tc_tiling_on_sc=False
        ),
        scratch_types=[
            pltpu.VMEM((INPUT_DIM, CHUNK), jnp.float32),
            pltpu.VMEM((HALF, CHUNK), jnp.int32),
            pltpu.VMEM((HALF, CHUNK), jnp.float32),
            pltpu.VMEM((16,), jnp.float32),
            pltpu.VMEM((CHUNK,), jnp.float32),
            pltpu.SemaphoreType.DMA,
        ],
    )


def _block_diag_weights(wseg):
    return (
        jnp.eye(HALF, dtype=jnp.float32)[:, :, None] * wseg[None]
    ).reshape(HALF, HALF * EMB)


@jax.jit
def kernel(x, tables, W, b):
    w = W[0]
    wcat = w[: NUM_CAT * EMB].reshape(NUM_CAT, EMB)
    wb = jnp.concatenate([w[NUM_CAT * EMB :], b, jnp.zeros((2,), jnp.float32)])
    xt = x.T

    proj = pl.pallas_call(
        _project_body,
        grid=(NBLK,),
        in_specs=[
            pl.BlockSpec((HALF, HALF * EMB), lambda i: (0, 0)),
            pl.BlockSpec((HALF * EMB, CBLK), lambda i: (0, i)),
        ],
        out_specs=pl.BlockSpec((HALF, CBLK), lambda i: (0, i)),
        out_shape=jax.ShapeDtypeStruct((HALF, VOCAB), jnp.float32),
    )

    halves = []
    for h in range(2):
        f0 = h * HALF
        tablesT = (
            tables[f0 : f0 + HALF].transpose(0, 2, 1).reshape(HALF * EMB, VOCAB)
        )
        p_flat = proj(_block_diag_weights(wcat[f0 : f0 + HALF]), tablesT)
        halves.append(p_flat.reshape(HALF * VOCAB))

    zero = jnp.zeros((B,), jnp.float32)
    partial = _make_gather(True)(xt, halves[0], zero, wb)
    out = _make_gather(False)(xt, halves[1], partial, wb)
    return out.reshape(B, 1)


# speedup vs baseline: 2.2375x; 2.2375x over previous
import functools

import jax
import jax.numpy as jnp
from jax import lax
from jax.experimental import pallas as pl
from jax.experimental.pallas import tpu as pltpu
from jax.experimental.pallas import tpu_sc as plsc

B = 16384
INPUT_DIM = 39
NUM_CAT = 26
VOCAB = 100000
EMB = 32
NUM_NUM = 13
HALF = NUM_CAT // 2

NC = 2
NS = 16
NW = NC * NS
SAMPLES_PER_TILE = B // NW
CHUNK = 128
NUM_CHUNKS = SAMPLES_PER_TILE // CHUNK
GROUPS = CHUNK // 16

CBLK = 4096
NBLK = (VOCAB + CBLK - 1) // CBLK


def _project_body(w_ref, t_ref, p_ref):
    p_ref[...] = jax.lax.dot_general(
        w_ref[...],
        t_ref[...],
        dimension_numbers=(((1,), (0,)), ((), ())),
        precision=jax.lax.Precision.DEFAULT,
        preferred_element_type=jnp.float32,
    )


def _gather_body(first, xt_hbm, p_hbm, pin_hbm, wb_hbm, out_hbm,
                 xv, idxv, gv, wv, outv, sem):
    wid = lax.axis_index("s") * NC + lax.axis_index("c")
    tile_base = pl.multiple_of(wid * SAMPLES_PER_TILE, SAMPLES_PER_TILE)
    col0 = NUM_NUM if first else NUM_NUM + HALF

    pltpu.sync_copy(wb_hbm, wv)

    def chunk_body(c, carry):
        base = pl.multiple_of(tile_base + c * CHUNK, CHUNK)

        pltpu.sync_copy(xt_hbm.at[:, pl.ds(base, CHUNK)], xv)
        if not first:
            pltpu.sync_copy(pin_hbm.at[pl.ds(base, CHUNK)], outv)

        def idx_body(f, carry):
            for g in range(GROUPS):
                v = xv[col0 + f, pl.ds(g * 16, 16)].astype(jnp.int32)
                idxv[f, pl.ds(g * 16, 16)] = v + f * VOCAB
            return carry

        lax.fori_loop(0, HALF, idx_body, 0)

        copies = [
            pltpu.async_copy(p_hbm.at[idxv.at[f]], gv.at[f], sem)
            for f in range(HALF)
        ]
        for cp in copies:
            cp.wait()

        wtail = wv[pl.ds(0, 16)]
        for g in range(GROUPS):
            acc = jnp.zeros((16,), jnp.float32)
            for f in range(HALF):
                acc = acc + gv[f, pl.ds(g * 16, 16)]
            if first:
                for i in range(NUM_NUM):
                    acc = acc + xv[i, pl.ds(g * 16, 16)] * wtail[i]
                outv[pl.ds(g * 16, 16)] = acc + wtail[NUM_NUM]
            else:
                outv[pl.ds(g * 16, 16)] = outv[pl.ds(g * 16, 16)] + acc

        pltpu.sync_copy(outv, out_hbm.at[pl.ds(base, CHUNK)])
        return carry

    lax.fori_loop(0, NUM_CHUNKS, chunk_body, 0)


def _make_gather(first):
    mesh = plsc.VectorSubcoreMesh(
        core_axis_name="c", subcore_axis_name="s", num_cores=NC, num_subcores=NS
    )
    return pl.kernel(
        functools.partial(_gather_body, first),
        out_type=jax.ShapeDtypeStruct((B,), jnp.float32),
        mesh=mesh,
        compiler_params=pltpu.CompilerParams(
            needs_layout_passes=False, use_tc_tiling_on_sc=False
        ),
        scratch_types=[
            pltpu.VMEM((INPUT_DIM, CHUNK), jnp.float32),
            pltpu.VMEM((HALF, CHUNK), jnp.int32),
            pltpu.VMEM((HALF, CHUNK), jnp.float32),
            pltpu.VMEM((16,), jnp.float32),
            pltpu.VMEM((CHUNK,), jnp.float32),
            pltpu.SemaphoreType.DMA,
        ],
    )


def _block_diag_weights(wseg):
    return (
        jnp.eye(HALF, dtype=jnp.float32)[:, :, None] * wseg[None]
    ).reshape(HALF, HALF * EMB)


@jax.jit
def kernel(x, tables, W, b):
    w = W[0]
    wcat = w[: NUM_CAT * EMB].reshape(NUM_CAT, EMB)
    wb = jnp.concatenate([w[NUM_CAT * EMB :], b, jnp.zeros((2,), jnp.float32)])
    xt = x.T

    tablesT = tables.transpose(0, 2, 1).reshape(NUM_CAT * EMB, VOCAB)

    halves = []
    for h in range(2):
        f0 = h * HALF
        proj = pl.pallas_call(
            _project_body,
            grid=(NBLK,),
            in_specs=[
                pl.BlockSpec((HALF, HALF * EMB), lambda i: (0, 0)),
                pl.BlockSpec((HALF * EMB, CBLK), lambda i, h=h: (h, i)),
            ],
            out_specs=pl.BlockSpec((HALF, CBLK), lambda i: (0, i)),
            out_shape=jax.ShapeDtypeStruct((HALF, VOCAB), jnp.float32),
        )
        p_flat = proj(_block_diag_weights(wcat[f0 : f0 + HALF]), tablesT)
        halves.append(p_flat.reshape(HALF * VOCAB))

    zero = jnp.zeros((B,), jnp.float32)
    partial = _make_gather(True)(xt, halves[0], zero, wb)
    out = _make_gather(False)(xt, halves[1], partial, wb)
    return out.reshape(B, 1)
